# Initial kernel scaffold; baseline (speedup 1.0000x reference)
import jax
import jax.numpy as jnp
from jax.experimental import pallas as pl


def kernel(x, edge_index, batch, W_tp, b_tp, Wl1, Wr1, att1, bias1, gn1_w, gn1_b, gn1_a, Wl4, Wr4, att4, bias4, gn4_w, gn4_b, gn4_a, W_lin, b_lin):
    raise NotImplementedError("write your pallas kernel here")


# speedup vs baseline: 18.0900x; 18.0900x over previous
import functools

import jax
import jax.numpy as jnp
from jax import lax
from jax.experimental import pallas as pl
from jax.experimental.pallas import tpu as pltpu
from jax.experimental.pallas import tpu_sc as plsc

N_NODES = 10000
N_PAD = 10240
T_SUB = 256
E_RAW = 320000
E_TOT = E_RAW + N_NODES
E_PAD = 331776
E_BLK = 1024
N_EBLK = E_PAD // E_BLK
CHUNK = 128
HEADS = 8
HC = 16
D1 = HEADS * HC
G = 64
NC_OUT = 7


def _sc_gather_pair(table_a, table_b, idx_a, idx_b, d):
    info = plsc.get_sparse_core_info()
    nw = info.num_cores * info.num_subcores
    b_per_w = E_PAD // nw
    n_chunks = b_per_w // CHUNK
    mesh = plsc.VectorSubcoreMesh(core_axis_name="c", subcore_axis_name="s")

    @functools.partial(
        pl.kernel,
        mesh=mesh,
        out_type=[
            jax.ShapeDtypeStruct((E_PAD, d), jnp.float32),
            jax.ShapeDtypeStruct((E_PAD, d), jnp.float32),
        ],
        scratch_types=[
            pltpu.VMEM((CHUNK,), jnp.int32),
            pltpu.VMEM((CHUNK, d), jnp.float32),
            pltpu.VMEM((CHUNK,), jnp.int32),
            pltpu.VMEM((CHUNK, d), jnp.float32),
            pltpu.SemaphoreType.DMA,
            pltpu.SemaphoreType.DMA,
        ],
    )
    def k(ta, tb, ia, ib, oa, ob, idx_va, rows_va, idx_vb, rows_vb, sem_a, sem_b):
        wid = lax.axis_index("s") * info.num_cores + lax.axis_index("c")
        base = wid * b_per_w

        def body(j, carry):
            off = base + j * CHUNK
            pltpu.sync_copy(ia.at[pl.ds(off, CHUNK)], idx_va)
            pltpu.sync_copy(ib.at[pl.ds(off, CHUNK)], idx_vb)
            cp_a = pltpu.async_copy(ta.at[idx_va], rows_va, sem_a)
            cp_b = pltpu.async_copy(tb.at[idx_vb], rows_vb, sem_b)
            cp_a.wait()
            cp_b.wait()
            pltpu.sync_copy(rows_va, oa.at[pl.ds(off, CHUNK)])
            pltpu.sync_copy(rows_vb, ob.at[pl.ds(off, CHUNK)])
            return carry

        lax.fori_loop(0, n_chunks, body, 0)

    return k(table_a, table_b, idx_a, idx_b)


def _proj_kernel(x_ref, wtp_ref, btp_ref, wl_ref, wr_ref, xl_ref, xr_ref):
    h = lax.dot_general(x_ref[...], wtp_ref[...], (((1,), (0,)), ((), ())),
                        preferred_element_type=jnp.float32) + btp_ref[...]
    xl_ref[...] = lax.dot_general(h, wl_ref[...], (((1,), (0,)), ((), ())),
                                  preferred_element_type=jnp.float32)
    xr_ref[...] = lax.dot_general(h, wr_ref[...], (((1,), (0,)), ((), ())),
                                  preferred_element_type=jnp.float32)


def _project(x_p, w_tp, b_tp, wl1, wr1):
    return pl.pallas_call(
        _proj_kernel,
        out_shape=[
            jax.ShapeDtypeStruct((N_PAD, D1), jnp.float32),
            jax.ShapeDtypeStruct((N_PAD, D1), jnp.float32),
        ],
    )(x_p, w_tp, b_tp, wl1, wr1)


def _head_matrix():
    k_i = lax.broadcasted_iota(jnp.int32, (D1, HEADS), 0)
    h_i = lax.broadcasted_iota(jnp.int32, (D1, HEADS), 1)
    return ((k_i // HC) == h_i).astype(jnp.float32)


def _edge_scatter1_kernel(nb_ref, cnt_ref, zl_ref, zr_ref, dst_ref, att_ref,
                          agg_ref, den_ref):
    b = pl.program_id(0)

    @pl.when(b == 0)
    def _init():
        agg_ref[...] = jnp.zeros_like(agg_ref)
        den_ref[...] = jnp.zeros_like(den_ref)

    zl = zl_ref[...]
    e = zl + zr_ref[...]
    e = jnp.where(e > 0, e, 0.2 * e)
    s_mat = _head_matrix()
    score = lax.dot_general(e * att_ref[...], s_mat, (((1,), (0,)), ((), ())),
                            preferred_element_type=jnp.float32)
    ex = jnp.exp(score)
    ex_wide = lax.dot_general(ex, s_mat, (((1,), (1,)), ((), ())),
                              preferred_element_type=jnp.float32)
    msg = zl * ex_wide
    d = dst_ref[0, 0, :]
    base = nb_ref[b]
    cnt = cnt_ref[b]

    def body(j, carry):
        node0 = (base + j) * T_SUB
        rows = lax.broadcasted_iota(jnp.int32, (T_SUB, E_BLK), 0) + node0
        oh = (rows == d[None, :]).astype(jnp.float32)
        agg_ref[pl.ds(node0, T_SUB), :] = agg_ref[pl.ds(node0, T_SUB), :] + \
            lax.dot_general(oh, msg, (((1,), (0,)), ((), ())),
                            preferred_element_type=jnp.float32)
        den_ref[pl.ds(node0, T_SUB), :] = den_ref[pl.ds(node0, T_SUB), :] + \
            lax.dot_general(oh, ex, (((1,), (0,)), ((), ())),
                            preferred_element_type=jnp.float32)
        return carry

    lax.fori_loop(0, cnt, body, 0)


def _edge_scatter1(nb, cnt, zl, zr, dst3, att_flat):
    grid_spec = pltpu.PrefetchScalarGridSpec(
        num_scalar_prefetch=2,
        grid=(N_EBLK,),
        in_specs=[
            pl.BlockSpec((E_BLK, D1), lambda b, *_: (b, 0)),
            pl.BlockSpec((E_BLK, D1), lambda b, *_: (b, 0)),
            pl.BlockSpec((1, 1, E_BLK), lambda b, *_: (b, 0, 0)),
            pl.BlockSpec((1, D1), lambda b, *_: (0, 0)),
        ],
        out_specs=[
            pl.BlockSpec((N_PAD, D1), lambda b, *_: (0, 0)),
            pl.BlockSpec((N_PAD, HEADS), lambda b, *_: (0, 0)),
        ],
    )
    return pl.pallas_call(
        _edge_scatter1_kernel,
        grid_spec=grid_spec,
        out_shape=[
            jax.ShapeDtypeStruct((N_PAD, D1), jnp.float32),
            jax.ShapeDtypeStruct((N_PAD, HEADS), jnp.float32),
        ],
    )(nb, cnt, zl, zr, dst3, att_flat)


def _edge_scatter4_kernel(nb_ref, cnt_ref, g1_ref, g2_ref, dst_ref, att_ref,
                          wl4_ref, wr4_ref, agg_ref, den_ref):
    b = pl.program_id(0)

    @pl.when(b == 0)
    def _init():
        agg_ref[...] = jnp.zeros_like(agg_ref)
        den_ref[...] = jnp.zeros_like(den_ref)

    zl = lax.dot_general(g1_ref[...], wl4_ref[...], (((1,), (0,)), ((), ())),
                         preferred_element_type=jnp.float32)
    zr = lax.dot_general(g2_ref[...], wr4_ref[...], (((1,), (0,)), ((), ())),
                         preferred_element_type=jnp.float32)
    e = zl + zr
    e = jnp.where(e > 0, e, 0.2 * e)
    s = jnp.sum(e * att_ref[...], axis=1, keepdims=True)
    ex = jnp.exp(s)
    msg = zl * ex
    ex8 = ex * jnp.ones((1, 8), jnp.float32)
    d = dst_ref[0, 0, :]
    base = nb_ref[b]
    cnt = cnt_ref[b]

    def body(j, carry):
        node0 = (base + j) * T_SUB
        rows = lax.broadcasted_iota(jnp.int32, (T_SUB, E_BLK), 0) + node0
        oh = (rows == d[None, :]).astype(jnp.float32)
        agg_ref[pl.ds(node0, T_SUB), :] = agg_ref[pl.ds(node0, T_SUB), :] + \
            lax.dot_general(oh, msg, (((1,), (0,)), ((), ())),
                            preferred_element_type=jnp.float32)
        den_ref[pl.ds(node0, T_SUB), :] = den_ref[pl.ds(node0, T_SUB), :] + \
            lax.dot_general(oh, ex8, (((1,), (0,)), ((), ())),
                            preferred_element_type=jnp.float32)
        return carry

    lax.fori_loop(0, cnt, body, 0)


def _edge_scatter4(nb, cnt, g1, g2, dst3, att_flat, wl4, wr4):
    grid_spec = pltpu.PrefetchScalarGridSpec(
        num_scalar_prefetch=2,
        grid=(N_EBLK,),
        in_specs=[
            pl.BlockSpec((E_BLK, D1), lambda b, *_: (b, 0)),
            pl.BlockSpec((E_BLK, D1), lambda b, *_: (b, 0)),
            pl.BlockSpec((1, 1, E_BLK), lambda b, *_: (b, 0, 0)),
            pl.BlockSpec((1, HC), lambda b, *_: (0, 0)),
            pl.BlockSpec((D1, HC), lambda b, *_: (0, 0)),
            pl.BlockSpec((D1, HC), lambda b, *_: (0, 0)),
        ],
        out_specs=[
            pl.BlockSpec((N_PAD, HC), lambda b, *_: (0, 0)),
            pl.BlockSpec((N_PAD, 8), lambda b, *_: (0, 0)),
        ],
    )
    return pl.pallas_call(
        _edge_scatter4_kernel,
        grid_spec=grid_spec,
        out_shape=[
            jax.ShapeDtypeStruct((N_PAD, HC), jnp.float32),
            jax.ShapeDtypeStruct((N_PAD, 8), jnp.float32),
        ],
    )(nb, cnt, g1, g2, dst3, att_flat, wl4, wr4)


def _graph_norm_block(x, oh_b, cnt_inv, w, bgn, a):
    mean = lax.dot_general(oh_b, x, (((0,), (0,)), ((), ())),
                           preferred_element_type=jnp.float32) * cnt_inv
    ctr = x - a * lax.dot_general(oh_b, mean, (((1,), (0,)), ((), ())),
                                  preferred_element_type=jnp.float32)
    var = lax.dot_general(oh_b, ctr * ctr, (((0,), (0,)), ((), ())),
                          preferred_element_type=jnp.float32) * cnt_inv
    std = jnp.sqrt(var + 1e-5)
    gstd = lax.dot_general(oh_b, std, (((1,), (0,)), ((), ())),
                           preferred_element_type=jnp.float32)
    gstd = jnp.where(gstd > 0, gstd, 1.0)
    return w * ctr / gstd + bgn


def _node1_kernel(agg_ref, den_ref, batch_ref, bias_ref, w_ref, bgn_ref, a_ref,
                  h1_ref):
    s_mat = _head_matrix()
    den_wide = lax.dot_general(den_ref[...], s_mat, (((1,), (1,)), ((), ())),
                               preferred_element_type=jnp.float32)
    out1 = agg_ref[...] / (den_wide + 1e-16) + bias_ref[...]
    bt = batch_ref[...]
    g_i = lax.broadcasted_iota(jnp.int32, (N_PAD, G), 1)
    oh_b = (bt == g_i).astype(jnp.float32)
    cnt = jnp.sum(oh_b, axis=0, keepdims=True)
    cnt_inv = (1.0 / jnp.maximum(cnt, 1.0)).reshape(G, 1)
    gn = _graph_norm_block(out1, oh_b, cnt_inv, w_ref[...], bgn_ref[...], a_ref[...])
    h1 = jnp.where(gn > 0, gn, jnp.exp(jnp.minimum(gn, 0.0)) - 1.0)
    h1_ref[...] = h1


def _node1(agg, den, batch2d, bias1, gn1_w, gn1_b, gn1_a):
    return pl.pallas_call(
        _node1_kernel,
        out_shape=jax.ShapeDtypeStruct((N_PAD, D1), jnp.float32),
    )(agg, den, batch2d, bias1, gn1_w, gn1_b, gn1_a)


def _node4_kernel(h1_ref, agg_ref, den_ref, batch_ref, bias_ref, w_ref, bgn_ref,
                  a_ref, wa_ref, wb_ref, blin_ref, out_ref):
    h4 = agg_ref[...] / (den_ref[...][:, :1] + 1e-16) + bias_ref[...]
    bt = batch_ref[...]
    g_i = lax.broadcasted_iota(jnp.int32, (N_PAD, G), 1)
    oh_b = (bt == g_i).astype(jnp.float32)
    cnt = jnp.sum(oh_b, axis=0, keepdims=True)
    cnt_inv = (1.0 / jnp.maximum(cnt, 1.0)).reshape(G, 1)
    h4n = _graph_norm_block(h4, oh_b, cnt_inv, w_ref[...], bgn_ref[...], a_ref[...])
    out = lax.dot_general(h1_ref[...], wa_ref[...], (((1,), (0,)), ((), ())),
                          preferred_element_type=jnp.float32)
    out = out + lax.dot_general(h4n, wb_ref[...], (((1,), (0,)), ((), ())),
                                preferred_element_type=jnp.float32)
    out_ref[...] = out + blin_ref[...]


def _node4(h1, agg4, den4, batch2d, bias4, gn4_w, gn4_b, gn4_a, w_a, w_b, b_lin):
    return pl.pallas_call(
        _node4_kernel,
        out_shape=jax.ShapeDtypeStruct((N_PAD, NC_OUT), jnp.float32),
    )(h1, agg4, den4, batch2d, bias4, gn4_w, gn4_b, gn4_a, w_a, w_b, b_lin)


def kernel(x, edge_index, batch, W_tp, b_tp, Wl1, Wr1, att1, bias1, gn1_w, gn1_b,
           gn1_a, Wl4, Wr4, att4, bias4, gn4_w, gn4_b, gn4_a, W_lin, b_lin):
    loop = jnp.arange(N_NODES, dtype=edge_index.dtype)
    src = jnp.concatenate([edge_index[0], loop])
    dst = jnp.concatenate([edge_index[1], loop])
    order = jnp.argsort(dst)
    ssrc = src[order]
    sdst = dst[order]
    pad = E_PAD - E_TOT
    sentinel = N_PAD - 1
    ssrc = jnp.concatenate([ssrc, jnp.full((pad,), sentinel, jnp.int32)])
    sdst = jnp.concatenate([sdst, jnp.full((pad,), sentinel, jnp.int32)])

    firsts = sdst[0::E_BLK]
    lasts = sdst[E_BLK - 1::E_BLK]
    nb = (firsts // T_SUB).astype(jnp.int32)
    cnt = (lasts // T_SUB - nb + 1).astype(jnp.int32)

    dst3 = sdst.reshape(N_EBLK, 1, E_BLK)
    x_p = jnp.pad(x, ((0, N_PAD - N_NODES), (0, 0)))
    batch2d = jnp.pad(batch, (0, N_PAD - N_NODES), constant_values=G).reshape(N_PAD, 1)

    xl1, xr1 = _project(x_p, W_tp, b_tp.reshape(1, HC), Wl1, Wr1)
    zl1, zr1 = _sc_gather_pair(xl1, xr1, ssrc, sdst, D1)
    agg1, den1 = _edge_scatter1(nb, cnt, zl1, zr1, dst3, att1.reshape(1, D1))
    h1 = _node1(agg1, den1, batch2d, bias1.reshape(1, D1),
                gn1_w.reshape(1, D1), gn1_b.reshape(1, D1), gn1_a.reshape(1, D1))
    g1, g2 = _sc_gather_pair(h1, h1, ssrc, sdst, D1)
    agg4, den4 = _edge_scatter4(nb, cnt, g1, g2, dst3, att4.reshape(1, HC),
                                Wl4, Wr4)
    out = _node4(h1, agg4, den4, batch2d, bias4.reshape(1, HC),
                 gn4_w.reshape(1, HC), gn4_b.reshape(1, HC), gn4_a.reshape(1, HC),
                 W_lin[:D1], W_lin[D1:], b_lin.reshape(1, NC_OUT))
    return out[:N_NODES]
